# baseline (speedup 1.0000x reference)
import functools

import jax
import jax.numpy as jnp
from jax import lax
from jax.experimental import pallas as pl
from jax.experimental.pallas import tpu as pltpu
from jax.experimental.pallas import tpu_sc as plsc

_NUM_EMBEDDINGS = 100000
_SEQ_LEN = 200
_EMB_DIM = 64
_BATCH = 4096

_NW = 32
_CH = 100
_CH_PAD = 104
_BATCH_PER_W = _BATCH // _NW
_CHUNKS_PER_W = 2 * _BATCH_PER_W


def _position_embedding_host():
    even_index = jnp.arange(0, _EMB_DIM, 2, dtype=jnp.float32)
    denominator = jnp.power(10000.0, even_index / _EMB_DIM)
    positions = jnp.arange(0, _SEQ_LEN, dtype=jnp.float32).reshape(_SEQ_LEN, 1)
    even_pe = jnp.sin(positions / denominator)
    odd_pe = jnp.cos(positions / denominator)
    stacked = jnp.stack([even_pe, odd_pe], axis=2)
    return stacked.reshape(_SEQ_LEN, _EMB_DIM)


def _sc_body(table_hbm, idx_hbm, pos_hbm, out_hbm,
             idx_v, pos_v, buf_a, buf_b, sem_a, sem_b):
    nc = 2
    wid = lax.axis_index("s") * nc + lax.axis_index("c")
    chunk0 = wid * _CHUNKS_PER_W
    batch0 = wid * _BATCH_PER_W
    last_even = _CHUNKS_PER_W - 2

    pltpu.sync_copy(idx_hbm.at[pl.ds(chunk0, _CHUNKS_PER_W)], idx_v)
    pltpu.sync_copy(pos_hbm, pos_v)

    def fire(g, buf, sem, half):
        poff = half * _CH

        def cp(r, c):
            for cidx in range(_EMB_DIM // 16):
                sl = pl.ds(cidx * 16, 16)
                buf[r, sl] = pos_v[poff + r, sl]
            return c

        lax.fori_loop(0, _CH, cp, 0, unroll=4)
        return pltpu.async_copy(table_hbm.at[idx_v.at[g]], buf, sem, add=True)

    fire(0, buf_a, sem_a, 0)

    def body(go, carry):
        g = 2 * go
        b = batch0 + go
        fire(g + 1, buf_b, sem_b, 1)
        pltpu.make_async_copy(table_hbm.at[idx_v.at[0]], buf_a, sem_a).wait()
        pltpu.sync_copy(buf_a.at[pl.ds(0, _CH)], out_hbm.at[b, pl.ds(0, _CH)])
        fire(jnp.minimum(g + 2, last_even), buf_a, sem_a, 0)
        pltpu.make_async_copy(table_hbm.at[idx_v.at[0]], buf_b, sem_b).wait()
        pltpu.sync_copy(buf_b.at[pl.ds(0, _CH)],
                        out_hbm.at[b, pl.ds(_CH, _CH)])
        return carry

    lax.fori_loop(0, _BATCH_PER_W, body, 0)
    pltpu.make_async_copy(table_hbm.at[idx_v.at[0]], buf_a, sem_a).wait()


@jax.jit
def kernel(input, table):
    pos = _position_embedding_host()
    table_view = jnp.pad(table, ((0, 0), (0, 128 - _EMB_DIM))).reshape(
        2 * _NUM_EMBEDDINGS, _EMB_DIM
    )
    idx_padded = jnp.pad(
        (input.astype(jnp.int32) * 2).reshape(_BATCH * 2, _CH),
        ((0, 0), (0, _CH_PAD - _CH)),
    )

    mesh = plsc.VectorSubcoreMesh(core_axis_name="c", subcore_axis_name="s")
    out = pl.kernel(
        _sc_body,
        out_type=jax.ShapeDtypeStruct((_BATCH, _SEQ_LEN, _EMB_DIM), jnp.float32),
        mesh=mesh,
        scratch_types=[
            pltpu.VMEM((_CHUNKS_PER_W, _CH_PAD), jnp.int32),
            pltpu.VMEM((_SEQ_LEN, _EMB_DIM), jnp.float32),
            pltpu.VMEM((_CH_PAD, _EMB_DIM), jnp.float32),
            pltpu.VMEM((_CH_PAD, _EMB_DIM), jnp.float32),
            pltpu.SemaphoreType.DMA,
            pltpu.SemaphoreType.DMA,
        ],
        compiler_params=pltpu.CompilerParams(use_tc_tiling_on_sc=False),
    )(table_view, idx_padded, pos)
    return out


# speedup vs baseline: 1.5871x; 1.5871x over previous
import functools

import jax
import jax.numpy as jnp
from jax import lax
from jax.experimental import pallas as pl
from jax.experimental.pallas import tpu as pltpu
from jax.experimental.pallas import tpu_sc as plsc

_NUM_EMBEDDINGS = 100000
_SEQ_LEN = 200
_EMB_DIM = 64
_BATCH = 4096

_NW = 32
_CH = 100
_CH_PAD = 104
_BATCH_PER_W = _BATCH // _NW
_CHUNKS_PER_W = 2 * _BATCH_PER_W


def _position_embedding_host():
    even_index = jnp.arange(0, _EMB_DIM, 2, dtype=jnp.float32)
    denominator = jnp.power(10000.0, even_index / _EMB_DIM)
    positions = jnp.arange(0, _SEQ_LEN, dtype=jnp.float32).reshape(_SEQ_LEN, 1)
    even_pe = jnp.sin(positions / denominator)
    odd_pe = jnp.cos(positions / denominator)
    stacked = jnp.stack([even_pe, odd_pe], axis=2)
    return stacked.reshape(_SEQ_LEN, _EMB_DIM)


def _sc_body(table_hbm, idx_hbm, pos_hbm, out_hbm,
             idx_v, pos_v, buf_a, buf_b, sem_a, sem_b):
    nc = 2
    wid = lax.axis_index("s") * nc + lax.axis_index("c")
    chunk0 = wid * _CHUNKS_PER_W
    batch0 = wid * _BATCH_PER_W
    last_even = _CHUNKS_PER_W - 2

    pltpu.sync_copy(idx_hbm.at[pl.ds(chunk0, _CHUNKS_PER_W)], idx_v)
    pltpu.sync_copy(pos_hbm, pos_v)

    def fire(g, buf, sem, half):
        poff = half * _CH

        def cp(r, c):
            for cidx in range(_EMB_DIM // 16):
                sl = pl.ds(cidx * 16, 16)
                buf[r, sl] = pos_v[poff + r, sl]
            return c

        lax.fori_loop(0, _CH, cp, 0, unroll=4)
        return pltpu.async_copy(table_hbm.at[idx_v.at[g]], buf, sem, add=True)

    fire(0, buf_a, sem_a, 0)

    def body(go, carry):
        g = 2 * go
        b = batch0 + go
        fire(g + 1, buf_b, sem_b, 1)
        pltpu.make_async_copy(table_hbm.at[idx_v.at[0]], buf_a, sem_a).wait()
        pltpu.sync_copy(buf_a, out_hbm.at[b, pl.ds(0, _CH)])
        fire(jnp.minimum(g + 2, last_even), buf_a, sem_a, 0)
        pltpu.make_async_copy(table_hbm.at[idx_v.at[0]], buf_b, sem_b).wait()
        pltpu.sync_copy(buf_b, out_hbm.at[b, pl.ds(_CH, _CH)])
        return carry

    lax.fori_loop(0, _BATCH_PER_W, body, 0)
    pltpu.make_async_copy(table_hbm.at[idx_v.at[0]], buf_a, sem_a).wait()


@jax.jit
def kernel(input, table):
    pos = _position_embedding_host()
    idx2d = input.reshape(_BATCH * 2, _CH)

    mesh = plsc.VectorSubcoreMesh(core_axis_name="c", subcore_axis_name="s")
    out = pl.kernel(
        _sc_body,
        out_type=jax.ShapeDtypeStruct((_BATCH, _SEQ_LEN, _EMB_DIM), jnp.float32),
        mesh=mesh,
        scratch_types=[
            pltpu.VMEM((_CHUNKS_PER_W, _CH), jnp.int32),
            pltpu.VMEM((_SEQ_LEN, _EMB_DIM), jnp.float32),
            pltpu.VMEM((_CH, _EMB_DIM), jnp.float32),
            pltpu.VMEM((_CH, _EMB_DIM), jnp.float32),
            pltpu.SemaphoreType.DMA,
            pltpu.SemaphoreType.DMA,
        ],
        compiler_params=pltpu.CompilerParams(use_tc_tiling_on_sc=False),
    )(table, idx2d, pos)
    return out


# speedup vs baseline: 1.9974x; 1.2586x over previous
import functools

import jax
import jax.numpy as jnp
from jax import lax
from jax.experimental import pallas as pl
from jax.experimental.pallas import tpu as pltpu
from jax.experimental.pallas import tpu_sc as plsc

_NUM_EMBEDDINGS = 100000
_SEQ_LEN = 200
_EMB_DIM = 64
_BATCH = 4096

_NW = 32
_CH = 100
_CH_PAD = 104
_BATCH_PER_W = _BATCH // _NW
_CHUNKS_PER_W = 2 * _BATCH_PER_W


def _position_embedding_host():
    even_index = jnp.arange(0, _EMB_DIM, 2, dtype=jnp.float32)
    denominator = jnp.power(10000.0, even_index / _EMB_DIM)
    positions = jnp.arange(0, _SEQ_LEN, dtype=jnp.float32).reshape(_SEQ_LEN, 1)
    even_pe = jnp.sin(positions / denominator)
    odd_pe = jnp.cos(positions / denominator)
    stacked = jnp.stack([even_pe, odd_pe], axis=2)
    return stacked.reshape(_SEQ_LEN, _EMB_DIM)


def _sc_body(table_hbm, idx_hbm, pos_hbm, out_hbm,
             idx_v, pos_sh, buf_a, buf_b, sem_a, sem_b, psem_a, psem_b):
    nc = 2
    sid = lax.axis_index("s")
    wid = sid * nc + lax.axis_index("c")
    chunk0 = wid * _CHUNKS_PER_W
    batch0 = wid * _BATCH_PER_W
    last_even = _CHUNKS_PER_W - 2

    @pl.when(sid == 0)
    def _():
        pltpu.sync_copy(pos_hbm, pos_sh)

    pltpu.sync_copy(idx_hbm.at[pl.ds(chunk0, _CHUNKS_PER_W)], idx_v)
    plsc.subcore_barrier()

    def prefill(buf, psem, half):
        pltpu.async_copy(pos_sh.at[pl.ds(half * _CH, _CH)], buf, psem)

    def fire(g, buf, sem, psem, half):
        pltpu.make_async_copy(
            pos_sh.at[pl.ds(half * _CH, _CH)], buf, psem).wait()
        return pltpu.async_copy(table_hbm.at[idx_v.at[g]], buf, sem, add=True)

    prefill(buf_a, psem_a, 0)
    prefill(buf_b, psem_b, 1)
    fire(0, buf_a, sem_a, psem_a, 0)

    def body(go, carry):
        g = 2 * go
        b = batch0 + go
        fire(g + 1, buf_b, sem_b, psem_b, 1)
        pltpu.make_async_copy(table_hbm.at[idx_v.at[0]], buf_a, sem_a).wait()
        pltpu.sync_copy(buf_a, out_hbm.at[b, pl.ds(0, _CH)])
        prefill(buf_a, psem_a, 0)
        fire(jnp.minimum(g + 2, last_even), buf_a, sem_a, psem_a, 0)
        pltpu.make_async_copy(table_hbm.at[idx_v.at[0]], buf_b, sem_b).wait()
        pltpu.sync_copy(buf_b, out_hbm.at[b, pl.ds(_CH, _CH)])
        prefill(buf_b, psem_b, 1)
        return carry

    lax.fori_loop(0, _BATCH_PER_W, body, 0)
    pltpu.make_async_copy(table_hbm.at[idx_v.at[0]], buf_a, sem_a).wait()
    pltpu.make_async_copy(pos_sh.at[pl.ds(_CH, _CH)], buf_b, psem_b).wait()


@jax.jit
def kernel(input, table):
    pos = _position_embedding_host()
    idx2d = input.reshape(_BATCH * 2, _CH)

    mesh = plsc.VectorSubcoreMesh(core_axis_name="c", subcore_axis_name="s")
    out = pl.kernel(
        _sc_body,
        out_type=jax.ShapeDtypeStruct((_BATCH, _SEQ_LEN, _EMB_DIM), jnp.float32),
        mesh=mesh,
        scratch_types=[
            pltpu.VMEM((_CHUNKS_PER_W, _CH), jnp.int32),
            pltpu.VMEM_SHARED((_SEQ_LEN, _EMB_DIM), jnp.float32),
            pltpu.VMEM((_CH, _EMB_DIM), jnp.float32),
            pltpu.VMEM((_CH, _EMB_DIM), jnp.float32),
            pltpu.SemaphoreType.DMA,
            pltpu.SemaphoreType.DMA,
            pltpu.SemaphoreType.DMA,
            pltpu.SemaphoreType.DMA,
        ],
        compiler_params=pltpu.CompilerParams(use_tc_tiling_on_sc=False),
    )(table, idx2d, pos)
    return out


# speedup vs baseline: 2.0209x; 1.0117x over previous
import functools

import jax
import jax.numpy as jnp
from jax import lax
from jax.experimental import pallas as pl
from jax.experimental.pallas import tpu as pltpu
from jax.experimental.pallas import tpu_sc as plsc

_NUM_EMBEDDINGS = 100000
_SEQ_LEN = 200
_EMB_DIM = 64
_BATCH = 4096

_NW = 32
_CH = 100
_CH_PAD = 104
_BATCH_PER_W = _BATCH // _NW
_CHUNKS_PER_W = 2 * _BATCH_PER_W


def _position_embedding_host():
    even_index = jnp.arange(0, _EMB_DIM, 2, dtype=jnp.float32)
    denominator = jnp.power(10000.0, even_index / _EMB_DIM)
    positions = jnp.arange(0, _SEQ_LEN, dtype=jnp.float32).reshape(_SEQ_LEN, 1)
    even_pe = jnp.sin(positions / denominator)
    odd_pe = jnp.cos(positions / denominator)
    stacked = jnp.stack([even_pe, odd_pe], axis=2)
    return stacked.reshape(_SEQ_LEN, _EMB_DIM)


def _sc_body(table_hbm, idx_hbm, pos_hbm, out_hbm,
             idx_v, pos_sh, buf_a, buf_b, sem_a, sem_b, psem_a, psem_b):
    nc = 2
    sid = lax.axis_index("s")
    wid = sid * nc + lax.axis_index("c")
    chunk0 = wid * _CHUNKS_PER_W
    batch0 = wid * _BATCH_PER_W
    last_even = _CHUNKS_PER_W - 2

    @pl.when(sid == 0)
    def _():
        pltpu.sync_copy(pos_hbm, pos_sh)

    pltpu.sync_copy(idx_hbm.at[pl.ds(chunk0, _CHUNKS_PER_W)], idx_v)
    plsc.subcore_barrier()

    def prefill(buf, psem, half):
        pltpu.async_copy(pos_sh.at[pl.ds(half * _CH, _CH)], buf, psem)

    def fire(g, buf, sem, psem, half):
        pltpu.make_async_copy(
            pos_sh.at[pl.ds(half * _CH, _CH)], buf, psem).wait()
        return pltpu.async_copy(table_hbm.at[idx_v.at[g]], buf, sem, add=True)

    prefill(buf_a, psem_a, 0)
    prefill(buf_b, psem_b, 1)
    fire(0, buf_a, sem_a, psem_a, 0)

    def body(go, carry):
        g = 2 * go
        b = batch0 + go
        fire(g + 1, buf_b, sem_b, psem_b, 1)
        pltpu.make_async_copy(table_hbm.at[idx_v.at[0]], buf_a, sem_a).wait()
        pltpu.sync_copy(buf_a, out_hbm.at[b, pl.ds(0, _CH)])
        prefill(buf_a, psem_a, 0)
        fire(jnp.minimum(g + 2, last_even), buf_a, sem_a, psem_a, 0)
        pltpu.make_async_copy(table_hbm.at[idx_v.at[0]], buf_b, sem_b).wait()
        pltpu.sync_copy(buf_b, out_hbm.at[b, pl.ds(_CH, _CH)])
        prefill(buf_b, psem_b, 1)
        return carry

    lax.fori_loop(0, _BATCH_PER_W, body, 0)
    pltpu.make_async_copy(table_hbm.at[idx_v.at[0]], buf_a, sem_a).wait()
    pltpu.make_async_copy(pos_sh.at[pl.ds(_CH, _CH)], buf_b, psem_b).wait()


@jax.jit
def kernel(input, table):
    pos = _position_embedding_host()
    table = jnp.pad(table, ((0, 0), (0, 128 - _EMB_DIM))).reshape(
        2 * _NUM_EMBEDDINGS, _EMB_DIM
    )
    idx2d = (input * 2).reshape(_BATCH * 2, _CH)

    mesh = plsc.VectorSubcoreMesh(core_axis_name="c", subcore_axis_name="s")
    out = pl.kernel(
        _sc_body,
        out_type=jax.ShapeDtypeStruct((_BATCH, _SEQ_LEN, _EMB_DIM), jnp.float32),
        mesh=mesh,
        scratch_types=[
            pltpu.VMEM((_CHUNKS_PER_W, _CH), jnp.int32),
            pltpu.VMEM_SHARED((_SEQ_LEN, _EMB_DIM), jnp.float32),
            pltpu.VMEM((_CH, _EMB_DIM), jnp.float32),
            pltpu.VMEM((_CH, _EMB_DIM), jnp.float32),
            pltpu.SemaphoreType.DMA,
            pltpu.SemaphoreType.DMA,
            pltpu.SemaphoreType.DMA,
            pltpu.SemaphoreType.DMA,
        ],
        compiler_params=pltpu.CompilerParams(use_tc_tiling_on_sc=False),
    )(table, idx2d, pos)
    return out
